# baseline (speedup 1.0000x reference)
import jax
import jax.numpy as jnp
from jax.experimental import pallas as pl

BINS = 10
EPS = 1e-12
DELTA = 1.0 / BINS
N = 4096
BI = 512


def _bce_with_logits(x, z):
    return jnp.maximum(x, 0.0) - x * z + jnp.log1p(jnp.exp(-jnp.abs(x)))


def _tc_body(xc_ref, zc_ref, xr_ref, zr_ref, out_ref):
    xc = xc_ref[...]
    zc = zc_ref[...]
    xr = xr_ref[...]
    zr = zr_ref[...]
    gc = jnp.abs(jax.nn.sigmoid(xc) - zc)
    gr = jnp.abs(jax.nn.sigmoid(xr) - zr)
    close = (jnp.abs(gc - gr) <= DELTA).astype(jnp.float32)
    cnt = jnp.sum(close, axis=1, keepdims=True)
    gd = cnt / DELTA
    beta = N / (gd + EPS)
    loss = _bce_with_logits(xc, zc)
    out_ref[...] = jnp.sum(beta * loss).reshape(1, 1, 1)


def kernel(logits, targets):
    xc = logits.reshape(N, 1)
    zc = targets.reshape(N, 1)
    xr = logits.reshape(1, N)
    zr = targets.reshape(1, N)
    partial = pl.pallas_call(
        _tc_body,
        grid=(N // BI,),
        in_specs=[
            pl.BlockSpec((BI, 1), lambda i: (i, 0)),
            pl.BlockSpec((BI, 1), lambda i: (i, 0)),
            pl.BlockSpec((1, N), lambda i: (0, 0)),
            pl.BlockSpec((1, N), lambda i: (0, 0)),
        ],
        out_specs=pl.BlockSpec((1, 1, 1), lambda i: (i, 0, 0)),
        out_shape=jax.ShapeDtypeStruct((N // BI, 1, 1), jnp.float32),
    )(xc, zc, xr, zr)
    return jnp.sum(partial) / N
